# baseline (speedup 1.0000x reference)
import functools

import jax
import jax.numpy as jnp
import numpy as np
from jax import lax
from jax.experimental import pallas as pl
from jax.experimental.pallas import tpu as pltpu
from jax.experimental.pallas import tpu_sc as plsc

E = 160000
N = 10000
D_X = 80
D_XP = 128
D_MSG = 240
D_MP = 256
BE = 2000
_EH = 80000

_NC = 2
_NS = 16
_NW = _NC * _NS
_EPT = E // _NW
_GCH = 200

_NPT = 320
_ACC2 = 328
_SEC = 4000
_NSEC = _EH // _SEC
_CH = 64
_CBUF = _SEC + 2 * _CH

_SQ2 = float(np.sqrt(2.0))
_SQ3 = float(np.sqrt(3.0))
_SQ8 = float(np.sqrt(8.0))


def _build_consts():
    T16_0 = np.zeros((4, 48), np.float32)
    TmS = np.zeros((4, 48), np.float32)
    TpS = np.zeros((4, 48), np.float32)
    R16_48 = np.zeros((16, 48), np.float32)
    C48 = np.zeros((48, 16), np.float32)
    Sp = np.zeros((48, 48), np.float32)
    Sm = np.zeros((48, 48), np.float32)
    for i in range(16):
        for k in range(3):
            T16_0[1 + k, 3 * i + k] = 1.0
            TmS[1 + (k + 2) % 3, 3 * i + k] = 1.0
            TpS[1 + (k + 1) % 3, 3 * i + k] = 1.0
            R16_48[i, 3 * i + k] = 1.0
            C48[3 * i + k, i] = 1.0 / _SQ3
            Sp[3 * i + (k + 1) % 3, 3 * i + k] = 1.0 / _SQ2
            Sm[3 * i + (k + 2) % 3, 3 * i + k] = 1.0 / _SQ2
    T32_96 = np.zeros((4, 96), np.float32)
    R32 = np.zeros((32, 96), np.float32)
    for i in range(32):
        for k in range(3):
            T32_96[1 + k, 3 * i + k] = 1.0
            R32[i, 3 * i + k] = 1.0
    Y0_32 = np.zeros((4, 32), np.float32)
    Y0_32[0, :] = 1.0
    Y0_48 = np.zeros((4, 48), np.float32)
    Y0_48[0, :] = 1.0
    SCAT = np.concatenate([Sp, Sm], axis=1)
    return T16_0, TmS, TpS, R16_48, C48, T32_96, R32, Y0_32, Y0_48, SCAT


_CONSTS = _build_consts()


def _tc_body(xs_ref, sph_ref, rbf_ref, W1_ref, W2_ref, W3p_ref,
             T16_0_ref, TmS_ref, TpS_ref, R16_48_ref, C48_ref, T32_96_ref,
             R32_ref, Y0_32_ref, Y0_48_ref, SCAT_ref,
             msg_ref, msgp_ref):
    f32 = jnp.float32
    dot = lambda a, b: jnp.dot(a, b, preferred_element_type=f32)
    rbf = rbf_ref[...]
    h = jnp.tanh(dot(rbf, W1_ref[...]) * (1.0 / _SQ8))
    h = jnp.tanh(dot(h, W2_ref[...]) * 0.125)
    w = dot(h, W3p_ref[...]) * 0.125
    wA = w[:, 0:32]
    wB = w[:, 128:144]
    wC = w[:, 256:288]
    wD = w[:, 384:400]
    wE = w[:, 512:528]
    xs = xs_ref[...]
    x0 = xs[:, 0:32]
    xv = xs[:, 32:80]
    sph = sph_ref[...]
    out0 = wA * x0 * dot(sph, Y0_32_ref[...])
    out1 = wB * dot(xv * dot(sph, T16_0_ref[...]), C48_ref[...])
    a2 = dot(wC * x0, R32_ref[...])
    out2 = a2 * dot(sph, T32_96_ref[...])
    out3 = dot(wD, R16_48_ref[...]) * xv * dot(sph, Y0_48_ref[...])
    xr = dot(xv, SCAT_ref[...])
    cross = (xr[:, 0:48] * dot(sph, TmS_ref[...])
             - xr[:, 48:96] * dot(sph, TpS_ref[...]))
    out4 = dot(wE, R16_48_ref[...]) * cross
    msg_ref[:, 0:32] = out0
    msg_ref[:, 32:48] = out1
    msg_ref[:, 48:144] = out2
    msg_ref[:, 144:192] = out3
    msg_ref[:, 192:240] = out4
    msgp_ref[:, 0:240] = msg_ref[...]
    msgp_ref[:, 240:256] = jnp.zeros((msg_ref.shape[0], 16), f32)


def _message_tc(xs, edge_sph, edge_rbf, W1, W2, W3, prev_msg=None,
                interpret=False):
    W3p = jnp.zeros((64, 640), jnp.float32)
    W3p = W3p.at[:, 0:32].set(W3[:, 0:32])
    W3p = W3p.at[:, 128:144].set(W3[:, 32:48])
    W3p = W3p.at[:, 256:288].set(W3[:, 48:80])
    W3p = W3p.at[:, 384:400].set(W3[:, 80:96])
    W3p = W3p.at[:, 512:528].set(W3[:, 96:112])
    consts = [jnp.asarray(c) for c in _CONSTS]
    full = lambda a: pl.BlockSpec(a.shape, lambda i: (0,) * a.ndim)
    ob = 0 if prev_msg is None else _EH // BE
    edge_spec = lambda width: pl.BlockSpec((BE, width), lambda i: (i + ob, 0))
    in_specs = [
        edge_spec(D_XP),
        edge_spec(4),
        edge_spec(8),
        full(W1), full(W2), full(W3p),
        *[full(c) for c in consts],
    ]
    operands = [xs, edge_sph, edge_rbf, W1, W2, W3p, *consts]
    aliases = {}
    if prev_msg is not None:
        in_specs.append(pl.BlockSpec((8, D_MSG), lambda i: (0, 0)))
        operands.append(prev_msg)
        aliases = {len(operands) - 1: 0}
    body = _tc_body if prev_msg is None else (
        lambda *refs: _tc_body(*refs[:-3], refs[-2], refs[-1]))
    return pl.pallas_call(
        body,
        grid=(_EH // BE,),
        in_specs=in_specs,
        out_specs=[
            pl.BlockSpec((BE, D_MSG), lambda i: (i + ob, 0)),
            pl.BlockSpec((BE, D_MP), lambda i: (i, 0)),
        ],
        out_shape=[
            jax.ShapeDtypeStruct((E, D_MSG), jnp.float32),
            jax.ShapeDtypeStruct((_EH, D_MP), jnp.float32),
        ],
        input_output_aliases=aliases,
        interpret=interpret,
    )(*operands)


def _sc_gather(x_pad, src):
    mesh = plsc.VectorSubcoreMesh(core_axis_name="c", subcore_axis_name="s")

    @functools.partial(
        pl.kernel,
        out_type=jax.ShapeDtypeStruct((E, D_XP), jnp.float32),
        mesh=mesh,
        compiler_params=pltpu.CompilerParams(needs_layout_passes=False),
        scratch_types=[
            pltpu.VMEM((_EPT,), jnp.int32),
            pltpu.VMEM((_GCH, D_XP), jnp.float32),
            pltpu.VMEM((_GCH, D_XP), jnp.float32),
            pltpu.SemaphoreType.DMA,
            pltpu.SemaphoreType.DMA,
        ],
    )
    def k(x_hbm, src_hbm, out_hbm, idx_v, buf0, buf1, sem0, sem1):
        wid = lax.axis_index("s") * _NC + lax.axis_index("c")
        base = wid * _EPT
        pltpu.sync_copy(src_hbm.at[pl.ds(base, _EPT)], idx_v)
        n_ch = _EPT // _GCH
        bufs = (buf0, buf1)
        sems = (sem0, sem1)
        descs = [None] * n_ch
        descs[0] = pltpu.async_copy(
            x_hbm.at[idx_v.at[pl.ds(0, _GCH)]], buf0, sem0)
        for j in range(n_ch):
            if j + 1 < n_ch:
                descs[j + 1] = pltpu.async_copy(
                    x_hbm.at[idx_v.at[pl.ds((j + 1) * _GCH, _GCH)]],
                    bufs[(j + 1) % 2], sems[(j + 1) % 2])
            descs[j].wait()
            pltpu.sync_copy(bufs[j % 2],
                            out_hbm.at[pl.ds(base + j * _GCH, _GCH)])

    return k(x_pad, src)


def _sc_scatter(msg_pad, dst, denom16):
    mesh = plsc.VectorSubcoreMesh(core_axis_name="c", subcore_axis_name="s")

    @functools.partial(
        pl.kernel,
        out_type=jax.ShapeDtypeStruct((N, D_MP), jnp.float32),
        mesh=mesh,
        compiler_params=pltpu.CompilerParams(needs_layout_passes=False),
        scratch_types=[
            pltpu.VMEM((_SEC,), jnp.int32),
            pltpu.VMEM((_CBUF,), jnp.int32),
            pltpu.VMEM((_CH,), jnp.int32),
            pltpu.VMEM((_CH,), jnp.int32),
            pltpu.VMEM((_CH,), jnp.int32),
            pltpu.VMEM((_CH,), jnp.int32),
            pltpu.VMEM((_CH, D_MP), jnp.float32),
            pltpu.VMEM((_CH, D_MP), jnp.float32),
            pltpu.VMEM((16,), jnp.float32),
            pltpu.VMEM((_ACC2, D_MP), jnp.float32),
            pltpu.SemaphoreType.DMA,
            pltpu.SemaphoreType.DMA,
        ],
    )
    def k(msg_hbm, dst_hbm, den_hbm, out_hbm,
          secbuf, cbuf, eid0, loc0, eid1, loc1, stag0, stag1, dref, acc,
          sem0, sem1):
        eids_bufs = (eid0, eid1)
        locs_bufs = (loc0, loc1)
        stags = (stag0, stag1)
        sems = (sem0, sem1)
        c = lax.axis_index("c")
        s = lax.axis_index("s")
        w = s * _NC + c
        node_base = w * _NPT
        my_npt = jnp.minimum(_NPT, N - node_base)
        lanes = lax.iota(jnp.int32, 16)
        lanes9 = lanes << 9

        def zrow(r, _):
            z = jnp.zeros((16,), jnp.float32)
            for v in range(D_MP // 16):
                acc[r, pl.ds(v * 16, 16)] = z
            return 0
        lax.fori_loop(0, _ACC2, zrow, 0)

        pltpu.sync_copy(den_hbm, dref)

        def section(sec, _):
            ebase = sec * _SEC
            pltpu.sync_copy(dst_hbm.at[pl.ds(ebase, _SEC)], secbuf)

            def comp(i, cnt):
                d = secbuf[pl.ds(i * 16, 16)]
                loc = d - node_base
                m = plsc.bitcast(loc, jnp.uint32) < plsc.bitcast(
                    jnp.broadcast_to(my_npt, (16,)), jnp.uint32)
                mi = m.astype(jnp.int32)
                csum = plsc.cumsum(mi)
                pos = jnp.where(m, cnt + csum - mi, _CBUF - 16 + lanes)
                pack = (lanes9 + ((ebase + i * 16) << 9)) | (loc & (512 - 1))
                plsc.store_scatter(cbuf, [pos], pack)
                return cnt + csum[15]
            scnt = lax.fori_loop(0, _SEC // 16, comp, jnp.int32(0))

            dum = (lanes << 9) | (_NPT + (lanes & 7))
            for t in range(2 * _CH // 16):
                cbuf[pl.ds(scnt + t * 16, 16)] = dum

            n_ch = (scnt + _CH - 1) // _CH

            def unpack_fire(j, p):
                for v in range(_CH // 16):
                    pk = cbuf[pl.ds(j * _CH + v * 16, 16)]
                    eids_bufs[p][pl.ds(v * 16, 16)] = (
                        lax.shift_right_logical(pk, 9))
                    locs_bufs[p][pl.ds(v * 16, 16)] = pk & (512 - 1)
                pltpu.async_copy(msg_hbm.at[eids_bufs[p]], stags[p], sems[p])

            def accumulate(p):
                def radd(g, _):
                    lrv = locs_bufs[p][pl.ds(g * 16, 16)]
                    for l in range(16):
                        lr = lrv[l]
                        for v in range(D_MP // 16):
                            sl = pl.ds(v * 16, 16)
                            plsc.addupdate(acc.at[lr, sl],
                                           stags[p][g * 16 + l, sl])
                    return 0
                lax.fori_loop(0, _CH // 16, radd, 0)

            @pl.when(n_ch > 0)
            def _():
                unpack_fire(0, 0)

                def chunk(j, _):
                    def phase(p):
                        @pl.when(j + 1 < n_ch)
                        def _():
                            unpack_fire(j + 1, 1 - p)
                        pltpu.make_async_copy(
                            msg_hbm.at[eids_bufs[p]], stags[p], sems[p]
                        ).wait()
                        accumulate(p)

                    @pl.when(lax.rem(j, 2) == 0)
                    def _():
                        phase(0)

                    @pl.when(lax.rem(j, 2) == 1)
                    def _():
                        phase(1)
                    return 0
                lax.fori_loop(0, n_ch, chunk, 0)
            return 0
        lax.fori_loop(0, _NSEC, section, 0)

        rcp = 1.0 / dref[...]

        def srow(r, _):
            for v in range(D_MP // 16):
                sl = pl.ds(v * 16, 16)
                acc[r, sl] = acc[r, sl] * rcp
            return 0
        lax.fori_loop(0, _NPT, srow, 0)

        @pl.when(my_npt == _NPT)
        def _():
            for off in range(0, _NPT, 64):
                pltpu.sync_copy(acc.at[pl.ds(off, 64)],
                                out_hbm.at[pl.ds(node_base + off, 64)])

        @pl.when(my_npt < _NPT)
        def _():
            pltpu.sync_copy(acc.at[pl.ds(0, 64)],
                            out_hbm.at[pl.ds(node_base, 64)])
            pltpu.sync_copy(acc.at[pl.ds(64, 16)],
                            out_hbm.at[pl.ds(node_base + 64, 16)])

    return k(msg_pad, dst, denom16)


def kernel(node_sph_embed, edge_sph, edge_rbf_ebd, edge_index, W1, W2, W3,
           denominator):
    nf, nall, _ = node_sph_embed.shape
    x = node_sph_embed.reshape(nf * nall, D_X)
    x_pad = jnp.pad(x, ((0, 0), (0, D_XP - D_X)))
    edge_src = edge_index[:, 1]
    edge_dst = edge_index[:, 0]
    denom16 = jnp.broadcast_to(denominator, (16,))
    xs = _sc_gather(x_pad, edge_src)
    msg1, msgp1 = _message_tc(xs, edge_sph, edge_rbf_ebd, W1, W2, W3)
    part1 = _sc_scatter(msgp1, edge_dst[:_EH], denom16)
    message, msgp2 = _message_tc(xs, edge_sph, edge_rbf_ebd, W1, W2, W3,
                                 prev_msg=msg1)
    part2 = _sc_scatter(msgp2, edge_dst[_EH:], denom16)
    out = (part1 + part2)[:, :D_MSG].reshape(nf, nall, D_MSG)
    return (out, message)


# speedup vs baseline: 1.0352x; 1.0352x over previous
import functools

import jax
import jax.numpy as jnp
import numpy as np
from jax import lax
from jax.experimental import pallas as pl
from jax.experimental.pallas import tpu as pltpu
from jax.experimental.pallas import tpu_sc as plsc

E = 160000
N = 10000
D_X = 80
D_XP = 128
D_MSG = 240
D_MP = 256
BE = 2000

_NC = 2
_NS = 16
_NW = _NC * _NS
_EPT = E // _NW
_GCH = 400

_NPT = 320
_ACC2 = 328
_SEC = 4000
_NSEC = E // _SEC
_CH = 64
_CBUF = _SEC + 2 * _CH

_SQ2 = float(np.sqrt(2.0))
_SQ3 = float(np.sqrt(3.0))
_SQ8 = float(np.sqrt(8.0))


def _build_consts():
    T16_0 = np.zeros((4, 48), np.float32)
    TmS = np.zeros((4, 48), np.float32)
    TpS = np.zeros((4, 48), np.float32)
    R16_48 = np.zeros((16, 48), np.float32)
    C48 = np.zeros((48, 16), np.float32)
    Sp = np.zeros((48, 48), np.float32)
    Sm = np.zeros((48, 48), np.float32)
    for i in range(16):
        for k in range(3):
            T16_0[1 + k, 3 * i + k] = 1.0
            TmS[1 + (k + 2) % 3, 3 * i + k] = 1.0
            TpS[1 + (k + 1) % 3, 3 * i + k] = 1.0
            R16_48[i, 3 * i + k] = 1.0
            C48[3 * i + k, i] = 1.0 / _SQ3
            Sp[3 * i + (k + 1) % 3, 3 * i + k] = 1.0 / _SQ2
            Sm[3 * i + (k + 2) % 3, 3 * i + k] = 1.0 / _SQ2
    T32_96 = np.zeros((4, 96), np.float32)
    R32 = np.zeros((32, 96), np.float32)
    for i in range(32):
        for k in range(3):
            T32_96[1 + k, 3 * i + k] = 1.0
            R32[i, 3 * i + k] = 1.0
    Y0_32 = np.zeros((4, 32), np.float32)
    Y0_32[0, :] = 1.0
    Y0_48 = np.zeros((4, 48), np.float32)
    Y0_48[0, :] = 1.0
    SCAT = np.concatenate([Sp, Sm], axis=1)
    return T16_0, TmS, TpS, R16_48, C48, T32_96, R32, Y0_32, Y0_48, SCAT


_CONSTS = _build_consts()


def _tc_body(xs_ref, sph_ref, rbf_ref, W1_ref, W2_ref, W3p_ref,
             T16_0_ref, TmS_ref, TpS_ref, R16_48_ref, C48_ref, T32_96_ref,
             R32_ref, Y0_32_ref, Y0_48_ref, SCAT_ref,
             msg_ref, msgp_ref):
    f32 = jnp.float32
    dot = lambda a, b: jnp.dot(a, b, preferred_element_type=f32)
    rbf = rbf_ref[...]
    h = jnp.tanh(dot(rbf, W1_ref[...]) * (1.0 / _SQ8))
    h = jnp.tanh(dot(h, W2_ref[...]) * 0.125)
    w = dot(h, W3p_ref[...]) * 0.125
    wA = w[:, 0:32]
    wB = w[:, 128:144]
    wC = w[:, 256:288]
    wD = w[:, 384:400]
    wE = w[:, 512:528]
    xs = xs_ref[...]
    x0 = xs[:, 0:32]
    xv = xs[:, 32:80]
    sph = sph_ref[...]
    out0 = wA * x0 * dot(sph, Y0_32_ref[...])
    out1 = wB * dot(xv * dot(sph, T16_0_ref[...]), C48_ref[...])
    a2 = dot(wC * x0, R32_ref[...])
    out2 = a2 * dot(sph, T32_96_ref[...])
    out3 = dot(wD, R16_48_ref[...]) * xv * dot(sph, Y0_48_ref[...])
    xr = dot(xv, SCAT_ref[...])
    cross = (xr[:, 0:48] * dot(sph, TmS_ref[...])
             - xr[:, 48:96] * dot(sph, TpS_ref[...]))
    out4 = dot(wE, R16_48_ref[...]) * cross
    msg_ref[:, 0:32] = out0
    msg_ref[:, 32:48] = out1
    msg_ref[:, 48:144] = out2
    msg_ref[:, 144:192] = out3
    msg_ref[:, 192:240] = out4
    msgp_ref[:, 0:240] = msg_ref[...]
    msgp_ref[:, 240:256] = jnp.zeros((msg_ref.shape[0], 16), f32)


def _message_tc(xs, edge_sph, edge_rbf, W1, W2, W3, interpret=False):
    W3p = jnp.zeros((64, 640), jnp.float32)
    W3p = W3p.at[:, 0:32].set(W3[:, 0:32])
    W3p = W3p.at[:, 128:144].set(W3[:, 32:48])
    W3p = W3p.at[:, 256:288].set(W3[:, 48:80])
    W3p = W3p.at[:, 384:400].set(W3[:, 80:96])
    W3p = W3p.at[:, 512:528].set(W3[:, 96:112])
    consts = [jnp.asarray(c) for c in _CONSTS]
    full = lambda a: pl.BlockSpec(a.shape, lambda i: (0,) * a.ndim)
    grid = (E // BE,)
    return pl.pallas_call(
        _tc_body,
        grid=grid,
        in_specs=[
            pl.BlockSpec((BE, D_XP), lambda i: (i, 0)),
            pl.BlockSpec((BE, 4), lambda i: (i, 0)),
            pl.BlockSpec((BE, 8), lambda i: (i, 0)),
            full(W1), full(W2), full(W3p),
            *[full(c) for c in consts],
        ],
        out_specs=[
            pl.BlockSpec((BE, D_MSG), lambda i: (i, 0)),
            pl.BlockSpec((BE, D_MP), lambda i: (i, 0)),
        ],
        out_shape=[
            jax.ShapeDtypeStruct((E, D_MSG), jnp.float32),
            jax.ShapeDtypeStruct((E, D_MP), jnp.float32),
        ],
        interpret=interpret,
    )(xs, edge_sph, edge_rbf, W1, W2, W3p, *consts)


def _sc_gather(x_pad, src):
    mesh = plsc.VectorSubcoreMesh(core_axis_name="c", subcore_axis_name="s")

    @functools.partial(
        pl.kernel,
        out_type=jax.ShapeDtypeStruct((E, D_XP), jnp.float32),
        mesh=mesh,
        compiler_params=pltpu.CompilerParams(needs_layout_passes=False),
        scratch_types=[
            pltpu.VMEM((_EPT,), jnp.int32),
            pltpu.VMEM((_GCH, D_XP), jnp.float32),
            pltpu.VMEM((_GCH, D_XP), jnp.float32),
            pltpu.SemaphoreType.DMA,
            pltpu.SemaphoreType.DMA,
        ],
    )
    def k(x_hbm, src_hbm, out_hbm, idx_v, buf0, buf1, sem0, sem1):
        wid = lax.axis_index("s") * _NC + lax.axis_index("c")
        base = wid * _EPT
        pltpu.sync_copy(src_hbm.at[pl.ds(base, _EPT)], idx_v)
        plan = []
        off = 0
        while off < _EPT:
            sz = min(_GCH, _EPT - off)
            plan.append((off, sz))
            off += sz
        bufs = (buf0, buf1)
        sems = (sem0, sem1)
        descs = [None] * len(plan)
        descs[0] = pltpu.async_copy(
            x_hbm.at[idx_v.at[pl.ds(0, plan[0][1])]],
            buf0.at[pl.ds(0, plan[0][1])], sem0)
        for j, (o, sz) in enumerate(plan):
            if j + 1 < len(plan):
                o2, sz2 = plan[j + 1]
                descs[j + 1] = pltpu.async_copy(
                    x_hbm.at[idx_v.at[pl.ds(o2, sz2)]],
                    bufs[(j + 1) % 2].at[pl.ds(0, sz2)],
                    sems[(j + 1) % 2])
            descs[j].wait()
            pltpu.sync_copy(bufs[j % 2].at[pl.ds(0, sz)],
                            out_hbm.at[pl.ds(base + o, sz)])

    return k(x_pad, src)


def _sc_scatter(msg_pad, dst, denom16):
    mesh = plsc.VectorSubcoreMesh(core_axis_name="c", subcore_axis_name="s")

    @functools.partial(
        pl.kernel,
        out_type=jax.ShapeDtypeStruct((N, D_MP), jnp.float32),
        mesh=mesh,
        compiler_params=pltpu.CompilerParams(needs_layout_passes=False),
        scratch_types=[
            pltpu.VMEM((_SEC,), jnp.int32),
            pltpu.VMEM((_CBUF,), jnp.int32),
            pltpu.VMEM((_CH,), jnp.int32),
            pltpu.VMEM((_CH,), jnp.int32),
            pltpu.VMEM((_CH,), jnp.int32),
            pltpu.VMEM((_CH,), jnp.int32),
            pltpu.VMEM((_CH, D_MP), jnp.float32),
            pltpu.VMEM((_CH, D_MP), jnp.float32),
            pltpu.VMEM((16,), jnp.float32),
            pltpu.VMEM((_ACC2, D_MP), jnp.float32),
            pltpu.SemaphoreType.DMA,
            pltpu.SemaphoreType.DMA,
        ],
    )
    def k(msg_hbm, dst_hbm, den_hbm, out_hbm,
          secbuf, cbuf, eid0, loc0, eid1, loc1, stag0, stag1, dref, acc,
          sem0, sem1):
        eids_bufs = (eid0, eid1)
        locs_bufs = (loc0, loc1)
        stags = (stag0, stag1)
        sems = (sem0, sem1)
        c = lax.axis_index("c")
        s = lax.axis_index("s")
        w = s * _NC + c
        node_base = w * _NPT
        my_npt = jnp.minimum(_NPT, N - node_base)
        lanes = lax.iota(jnp.int32, 16)
        lanes9 = lanes << 9

        def zrow(r, _):
            z = jnp.zeros((16,), jnp.float32)
            for v in range(D_MP // 16):
                acc[r, pl.ds(v * 16, 16)] = z
            return 0
        lax.fori_loop(0, _ACC2, zrow, 0)

        pltpu.sync_copy(den_hbm, dref)

        def section(sec, _):
            ebase = sec * _SEC
            pltpu.sync_copy(dst_hbm.at[pl.ds(ebase, _SEC)], secbuf)

            def comp(i, cnt):
                d = secbuf[pl.ds(i * 16, 16)]
                loc = d - node_base
                m = plsc.bitcast(loc, jnp.uint32) < plsc.bitcast(
                    jnp.broadcast_to(my_npt, (16,)), jnp.uint32)
                mi = m.astype(jnp.int32)
                csum = plsc.cumsum(mi)
                pos = jnp.where(m, cnt + csum - mi, _CBUF - 16 + lanes)
                pack = (lanes9 + ((ebase + i * 16) << 9)) | (loc & (512 - 1))
                plsc.store_scatter(cbuf, [pos], pack)
                return cnt + csum[15]
            scnt = lax.fori_loop(0, _SEC // 16, comp, jnp.int32(0))

            dum = (lanes << 9) | (_NPT + (lanes & 7))
            for t in range(2 * _CH // 16):
                cbuf[pl.ds(scnt + t * 16, 16)] = dum

            n_ch = (scnt + _CH - 1) // _CH

            def unpack_fire(j, p):
                for v in range(_CH // 16):
                    pk = cbuf[pl.ds(j * _CH + v * 16, 16)]
                    eids_bufs[p][pl.ds(v * 16, 16)] = (
                        lax.shift_right_logical(pk, 9))
                    locs_bufs[p][pl.ds(v * 16, 16)] = pk & (512 - 1)
                pltpu.async_copy(msg_hbm.at[eids_bufs[p]], stags[p], sems[p])

            def accumulate(p):
                def radd(g, _):
                    lrv = locs_bufs[p][pl.ds(g * 16, 16)]
                    for l in range(16):
                        lr = lrv[l]
                        for v in range(D_MP // 16):
                            sl = pl.ds(v * 16, 16)
                            plsc.addupdate(acc.at[lr, sl],
                                           stags[p][g * 16 + l, sl])
                    return 0
                lax.fori_loop(0, _CH // 16, radd, 0)

            @pl.when(n_ch > 0)
            def _():
                unpack_fire(0, 0)

                def chunk(j, _):
                    def phase(p):
                        @pl.when(j + 1 < n_ch)
                        def _():
                            unpack_fire(j + 1, 1 - p)
                        pltpu.make_async_copy(
                            msg_hbm.at[eids_bufs[p]], stags[p], sems[p]
                        ).wait()
                        accumulate(p)

                    @pl.when(lax.rem(j, 2) == 0)
                    def _():
                        phase(0)

                    @pl.when(lax.rem(j, 2) == 1)
                    def _():
                        phase(1)
                    return 0
                lax.fori_loop(0, n_ch, chunk, 0)
            return 0
        lax.fori_loop(0, _NSEC, section, 0)

        rcp = 1.0 / dref[...]

        def srow(r, _):
            for v in range(D_MP // 16):
                sl = pl.ds(v * 16, 16)
                acc[r, sl] = acc[r, sl] * rcp
            return 0
        lax.fori_loop(0, _NPT, srow, 0)

        @pl.when(my_npt == _NPT)
        def _():
            for off in range(0, _NPT, 64):
                pltpu.sync_copy(acc.at[pl.ds(off, 64)],
                                out_hbm.at[pl.ds(node_base + off, 64)])

        @pl.when(my_npt < _NPT)
        def _():
            pltpu.sync_copy(acc.at[pl.ds(0, 64)],
                            out_hbm.at[pl.ds(node_base, 64)])
            pltpu.sync_copy(acc.at[pl.ds(64, 16)],
                            out_hbm.at[pl.ds(node_base + 64, 16)])

    return k(msg_pad, dst, denom16)


def kernel(node_sph_embed, edge_sph, edge_rbf_ebd, edge_index, W1, W2, W3,
           denominator):
    nf, nall, _ = node_sph_embed.shape
    x = node_sph_embed.reshape(nf * nall, D_X)
    x_pad = jnp.pad(x, ((0, 0), (0, D_XP - D_X)))
    edge_src = edge_index[:, 1]
    edge_dst = edge_index[:, 0]
    xs = _sc_gather(x_pad, edge_src)
    message, msg_pad = _message_tc(xs, edge_sph, edge_rbf_ebd, W1, W2, W3)
    denom16 = jnp.broadcast_to(denominator, (16,))
    out_pad = _sc_scatter(msg_pad, edge_dst, denom16)
    out = out_pad[:, :D_MSG].reshape(nf, nall, D_MSG)
    return (out, message)


# speedup vs baseline: 1.0353x; 1.0002x over previous
import functools

import jax
import jax.numpy as jnp
import numpy as np
from jax import lax
from jax.experimental import pallas as pl
from jax.experimental.pallas import tpu as pltpu
from jax.experimental.pallas import tpu_sc as plsc

E = 160000
N = 10000
D_X = 80
D_XP = 128
D_MSG = 240
D_MP = 256
BE = 2000

_NC = 2
_NS = 16
_NW = _NC * _NS
_EPT = E // _NW
_GCH = 400

_NPT = 320
_ACC2 = 328
_SEC = 4000
_NSEC = E // _SEC
_CH = 64
_CBUF = _SEC + 2 * _CH

_SQ2 = float(np.sqrt(2.0))
_SQ3 = float(np.sqrt(3.0))
_SQ8 = float(np.sqrt(8.0))


def _build_consts():
    T16_0 = np.zeros((4, 48), np.float32)
    TmS = np.zeros((4, 48), np.float32)
    TpS = np.zeros((4, 48), np.float32)
    R16_48 = np.zeros((16, 48), np.float32)
    C48 = np.zeros((48, 16), np.float32)
    Sp = np.zeros((48, 48), np.float32)
    Sm = np.zeros((48, 48), np.float32)
    for i in range(16):
        for k in range(3):
            T16_0[1 + k, 3 * i + k] = 1.0
            TmS[1 + (k + 2) % 3, 3 * i + k] = 1.0
            TpS[1 + (k + 1) % 3, 3 * i + k] = 1.0
            R16_48[i, 3 * i + k] = 1.0
            C48[3 * i + k, i] = 1.0 / _SQ3
            Sp[3 * i + (k + 1) % 3, 3 * i + k] = 1.0 / _SQ2
            Sm[3 * i + (k + 2) % 3, 3 * i + k] = 1.0 / _SQ2
    T32_96 = np.zeros((4, 96), np.float32)
    R32 = np.zeros((32, 96), np.float32)
    for i in range(32):
        for k in range(3):
            T32_96[1 + k, 3 * i + k] = 1.0
            R32[i, 3 * i + k] = 1.0
    Y0_32 = np.zeros((4, 32), np.float32)
    Y0_32[0, :] = 1.0
    Y0_48 = np.zeros((4, 48), np.float32)
    Y0_48[0, :] = 1.0
    SCAT = np.concatenate([Sp, Sm], axis=1)
    return T16_0, TmS, TpS, R16_48, C48, T32_96, R32, Y0_32, Y0_48, SCAT


_CONSTS = _build_consts()


def _tc_body(xs_ref, sph_ref, rbf_ref, W1_ref, W2_ref, W3p_ref,
             T16_0_ref, TmS_ref, TpS_ref, R16_48_ref, C48_ref, T32_96_ref,
             R32_ref, Y0_32_ref, Y0_48_ref, SCAT_ref,
             msg_ref, msgp_ref):
    f32 = jnp.float32
    dot = lambda a, b: jnp.dot(a, b, preferred_element_type=f32)
    rbf = rbf_ref[...]
    h = jnp.tanh(dot(rbf, W1_ref[...]) * (1.0 / _SQ8))
    h = jnp.tanh(dot(h, W2_ref[...]) * 0.125)
    w = dot(h, W3p_ref[...]) * 0.125
    wA = w[:, 0:32]
    wB = w[:, 128:144]
    wC = w[:, 256:288]
    wD = w[:, 384:400]
    wE = w[:, 512:528]
    xs = xs_ref[...]
    x0 = xs[:, 0:32]
    xv = xs[:, 32:80]
    sph = sph_ref[...]
    out0 = wA * x0 * dot(sph, Y0_32_ref[...])
    out1 = wB * dot(xv * dot(sph, T16_0_ref[...]), C48_ref[...])
    a2 = dot(wC * x0, R32_ref[...])
    out2 = a2 * dot(sph, T32_96_ref[...])
    out3 = dot(wD, R16_48_ref[...]) * xv * dot(sph, Y0_48_ref[...])
    xr = dot(xv, SCAT_ref[...])
    cross = (xr[:, 0:48] * dot(sph, TmS_ref[...])
             - xr[:, 48:96] * dot(sph, TpS_ref[...]))
    out4 = dot(wE, R16_48_ref[...]) * cross
    msg_ref[:, 0:32] = out0
    msg_ref[:, 32:48] = out1
    msg_ref[:, 48:144] = out2
    msg_ref[:, 144:192] = out3
    msg_ref[:, 192:240] = out4
    msgp_ref[:, 0:240] = msg_ref[...]
    msgp_ref[:, 240:256] = jnp.zeros((msg_ref.shape[0], 16), f32)


def _message_tc(xs, edge_sph, edge_rbf, W1, W2, W3, interpret=False):
    W3p = jnp.zeros((64, 640), jnp.float32)
    W3p = W3p.at[:, 0:32].set(W3[:, 0:32])
    W3p = W3p.at[:, 128:144].set(W3[:, 32:48])
    W3p = W3p.at[:, 256:288].set(W3[:, 48:80])
    W3p = W3p.at[:, 384:400].set(W3[:, 80:96])
    W3p = W3p.at[:, 512:528].set(W3[:, 96:112])
    consts = [jnp.asarray(c) for c in _CONSTS]
    full = lambda a: pl.BlockSpec(a.shape, lambda i: (0,) * a.ndim)
    grid = (E // BE,)
    return pl.pallas_call(
        _tc_body,
        grid=grid,
        in_specs=[
            pl.BlockSpec((BE, D_XP), lambda i: (i, 0)),
            pl.BlockSpec((BE, 4), lambda i: (i, 0)),
            pl.BlockSpec((BE, 8), lambda i: (i, 0)),
            full(W1), full(W2), full(W3p),
            *[full(c) for c in consts],
        ],
        out_specs=[
            pl.BlockSpec((BE, D_MSG), lambda i: (i, 0)),
            pl.BlockSpec((BE, D_MP), lambda i: (i, 0)),
        ],
        out_shape=[
            jax.ShapeDtypeStruct((E, D_MSG), jnp.float32),
            jax.ShapeDtypeStruct((E, D_MP), jnp.float32),
        ],
        interpret=interpret,
    )(xs, edge_sph, edge_rbf, W1, W2, W3p, *consts)


def _sc_gather(x_pad, src):
    mesh = plsc.VectorSubcoreMesh(core_axis_name="c", subcore_axis_name="s")

    @functools.partial(
        pl.kernel,
        out_type=jax.ShapeDtypeStruct((E, D_XP), jnp.float32),
        mesh=mesh,
        compiler_params=pltpu.CompilerParams(needs_layout_passes=False),
        scratch_types=[
            pltpu.VMEM((_EPT,), jnp.int32),
            pltpu.VMEM((_GCH, D_XP), jnp.float32),
            pltpu.VMEM((_GCH, D_XP), jnp.float32),
            pltpu.SemaphoreType.DMA,
            pltpu.SemaphoreType.DMA,
        ],
    )
    def k(x_hbm, src_hbm, out_hbm, idx_v, buf0, buf1, sem0, sem1):
        wid = lax.axis_index("s") * _NC + lax.axis_index("c")
        base = wid * _EPT
        pltpu.sync_copy(src_hbm.at[pl.ds(base, _EPT)], idx_v)
        plan = []
        off = 0
        while off < _EPT:
            sz = min(_GCH, _EPT - off)
            plan.append((off, sz))
            off += sz
        bufs = (buf0, buf1)
        sems = (sem0, sem1)
        descs = [None] * len(plan)
        descs[0] = pltpu.async_copy(
            x_hbm.at[idx_v.at[pl.ds(0, plan[0][1])]],
            buf0.at[pl.ds(0, plan[0][1])], sem0)
        for j, (o, sz) in enumerate(plan):
            if j + 1 < len(plan):
                o2, sz2 = plan[j + 1]
                descs[j + 1] = pltpu.async_copy(
                    x_hbm.at[idx_v.at[pl.ds(o2, sz2)]],
                    bufs[(j + 1) % 2].at[pl.ds(0, sz2)],
                    sems[(j + 1) % 2])
            descs[j].wait()
            pltpu.sync_copy(bufs[j % 2].at[pl.ds(0, sz)],
                            out_hbm.at[pl.ds(base + o, sz)])

    return k(x_pad, src)


def _sc_scatter(msg_pad, dst, denom16):
    mesh = plsc.VectorSubcoreMesh(core_axis_name="c", subcore_axis_name="s")

    @functools.partial(
        pl.kernel,
        out_type=jax.ShapeDtypeStruct((N, D_MP), jnp.float32),
        mesh=mesh,
        compiler_params=pltpu.CompilerParams(needs_layout_passes=False),
        scratch_types=[
            pltpu.VMEM((_SEC,), jnp.int32),
            pltpu.VMEM((_CBUF,), jnp.int32),
            pltpu.VMEM((_CH,), jnp.int32),
            pltpu.VMEM((_CH,), jnp.int32),
            pltpu.VMEM((_CH,), jnp.int32),
            pltpu.VMEM((_CH,), jnp.int32),
            pltpu.VMEM((_CH, D_MP), jnp.float32),
            pltpu.VMEM((_CH, D_MP), jnp.float32),
            pltpu.VMEM((16,), jnp.float32),
            pltpu.VMEM((_ACC2, D_MP), jnp.float32),
            pltpu.SemaphoreType.DMA,
            pltpu.SemaphoreType.DMA,
        ],
    )
    def k(msg_hbm, dst_hbm, den_hbm, out_hbm,
          secbuf, cbuf, eid0, loc0, eid1, loc1, stag0, stag1, dref, acc,
          sem0, sem1):
        eids_bufs = (eid0, eid1)
        locs_bufs = (loc0, loc1)
        stags = (stag0, stag1)
        sems = (sem0, sem1)
        c = lax.axis_index("c")
        s = lax.axis_index("s")
        w = s * _NC + c
        node_base = w * _NPT
        my_npt = jnp.minimum(_NPT, N - node_base)
        lanes = lax.iota(jnp.int32, 16)
        lanes9 = lanes << 9

        def zrow(r, _):
            z = jnp.zeros((16,), jnp.float32)
            for v in range(D_MP // 16):
                acc[r, pl.ds(v * 16, 16)] = z
            return 0
        lax.fori_loop(0, _ACC2, zrow, 0)

        pltpu.sync_copy(den_hbm, dref)

        def section(sec, _):
            ebase = sec * _SEC
            pltpu.sync_copy(dst_hbm.at[pl.ds(ebase, _SEC)], secbuf)

            def comp(i, cnt):
                for u in range(2):
                    d = secbuf[pl.ds(i * 32 + u * 16, 16)]
                    loc = d - node_base
                    m = plsc.bitcast(loc, jnp.uint32) < plsc.bitcast(
                        jnp.broadcast_to(my_npt, (16,)), jnp.uint32)
                    mi = m.astype(jnp.int32)
                    csum = plsc.cumsum(mi)
                    pos = jnp.where(m, cnt + csum - mi, _CBUF - 16 + lanes)
                    pack = ((lanes9 + ((ebase + i * 32 + u * 16) << 9))
                            | (loc & (512 - 1)))
                    plsc.store_scatter(cbuf, [pos], pack)
                    cnt = cnt + csum[15]
                return cnt
            scnt = lax.fori_loop(0, _SEC // 32, comp, jnp.int32(0))

            dum = (lanes << 9) | (_NPT + (lanes & 7))
            for t in range(2 * _CH // 16):
                cbuf[pl.ds(scnt + t * 16, 16)] = dum

            n_ch = (scnt + _CH - 1) // _CH

            def unpack_fire(j, p):
                for v in range(_CH // 16):
                    pk = cbuf[pl.ds(j * _CH + v * 16, 16)]
                    eids_bufs[p][pl.ds(v * 16, 16)] = (
                        lax.shift_right_logical(pk, 9))
                    locs_bufs[p][pl.ds(v * 16, 16)] = pk & (512 - 1)
                pltpu.async_copy(msg_hbm.at[eids_bufs[p]], stags[p], sems[p])

            def accumulate(p):
                def radd(g, _):
                    lrv = locs_bufs[p][pl.ds(g * 16, 16)]
                    for l in range(16):
                        lr = lrv[l]
                        for v in range(D_MP // 16):
                            sl = pl.ds(v * 16, 16)
                            plsc.addupdate(acc.at[lr, sl],
                                           stags[p][g * 16 + l, sl])
                    return 0
                lax.fori_loop(0, _CH // 16, radd, 0)

            @pl.when(n_ch > 0)
            def _():
                unpack_fire(0, 0)

                def chunk(j, _):
                    def phase(p):
                        @pl.when(j + 1 < n_ch)
                        def _():
                            unpack_fire(j + 1, 1 - p)
                        pltpu.make_async_copy(
                            msg_hbm.at[eids_bufs[p]], stags[p], sems[p]
                        ).wait()
                        accumulate(p)

                    @pl.when(lax.rem(j, 2) == 0)
                    def _():
                        phase(0)

                    @pl.when(lax.rem(j, 2) == 1)
                    def _():
                        phase(1)
                    return 0
                lax.fori_loop(0, n_ch, chunk, 0)
            return 0
        lax.fori_loop(0, _NSEC, section, 0)

        rcp = 1.0 / dref[...]

        def srow(r, _):
            for v in range(D_MP // 16):
                sl = pl.ds(v * 16, 16)
                acc[r, sl] = acc[r, sl] * rcp
            return 0
        lax.fori_loop(0, _NPT, srow, 0)

        @pl.when(my_npt == _NPT)
        def _():
            for off in range(0, _NPT, 64):
                pltpu.sync_copy(acc.at[pl.ds(off, 64)],
                                out_hbm.at[pl.ds(node_base + off, 64)])

        @pl.when(my_npt < _NPT)
        def _():
            pltpu.sync_copy(acc.at[pl.ds(0, 64)],
                            out_hbm.at[pl.ds(node_base, 64)])
            pltpu.sync_copy(acc.at[pl.ds(64, 16)],
                            out_hbm.at[pl.ds(node_base + 64, 16)])

    return k(msg_pad, dst, denom16)


def kernel(node_sph_embed, edge_sph, edge_rbf_ebd, edge_index, W1, W2, W3,
           denominator):
    nf, nall, _ = node_sph_embed.shape
    x = node_sph_embed.reshape(nf * nall, D_X)
    x_pad = jnp.pad(x, ((0, 0), (0, D_XP - D_X)))
    edge_src = edge_index[:, 1]
    edge_dst = edge_index[:, 0]
    xs = _sc_gather(x_pad, edge_src)
    message, msg_pad = _message_tc(xs, edge_sph, edge_rbf_ebd, W1, W2, W3)
    denom16 = jnp.broadcast_to(denominator, (16,))
    out_pad = _sc_scatter(msg_pad, edge_dst, denom16)
    out = out_pad[:, :D_MSG].reshape(nf, nall, D_MSG)
    return (out, message)
